# Initial kernel scaffold; baseline (speedup 1.0000x reference)
import jax
import jax.numpy as jnp
from jax.experimental import pallas as pl


def kernel(t, embeddings):
    raise NotImplementedError("write your pallas kernel here")


# speedup vs baseline: 5.5289x; 5.5289x over previous
import jax
import jax.numpy as jnp
from jax.experimental import pallas as pl
from jax.experimental.pallas import tpu as pltpu
from jax.experimental.pallas import tpu_sc as plsc

_WINDOW = 128


def kernel(t, embeddings):
    B, H = t.shape
    V, D = embeddings.shape
    N = B * H
    assert N % _WINDOW == 0

    idx = t.reshape(1, N).astype(jnp.int32)
    mesh = plsc.VectorSubcoreMesh(core_axis_name="core", subcore_axis_name="subcore")

    @pl.kernel(
        out_type=jax.ShapeDtypeStruct((N, D), embeddings.dtype),
        mesh=mesh,
        compiler_params=pltpu.CompilerParams(use_tc_tiling_on_sc=False),
    )
    def gather_kernel(x_hbm, i_hbm, o_hbm):
        def body(i_vmem, o_vmem):
            pltpu.sync_copy(x_hbm.at[i_vmem.at[0]], o_vmem)

        pltpu.emit_pipeline(
            body,
            grid=(N // _WINDOW,),
            in_specs=[pl.BlockSpec((1, _WINDOW), index_map=lambda i: (0, i))],
            out_specs=[pl.BlockSpec((_WINDOW, D), index_map=lambda i: (i, 0))],
            core_axis_name=("core", "subcore"),
            dimension_semantics=(pltpu.PARALLEL,),
        )(i_hbm, o_hbm)

    out = gather_kernel(embeddings, idx)
    return out.reshape(B, H, D)


# speedup vs baseline: 6.2036x; 1.1220x over previous
import jax
import jax.numpy as jnp
from jax.experimental import pallas as pl
from jax.experimental.pallas import tpu as pltpu
from jax.experimental.pallas import tpu_sc as plsc

_W = 128
_K = 4


def kernel(t, embeddings):
    B, H = t.shape
    V, D = embeddings.shape
    N = B * H
    assert N % (_K * _W) == 0

    idx = t.reshape(N // _W, _W).astype(jnp.int32)
    mesh = plsc.VectorSubcoreMesh(core_axis_name="core", subcore_axis_name="subcore")

    @pl.kernel(
        out_type=jax.ShapeDtypeStruct((N, D), embeddings.dtype),
        mesh=mesh,
        compiler_params=pltpu.CompilerParams(use_tc_tiling_on_sc=False),
    )
    def gather_kernel(x_hbm, i_hbm, o_hbm):
        def body(i_vmem, o_vmem):
            def scoped(sem):
                copies = [
                    pltpu.async_copy(
                        x_hbm.at[i_vmem.at[j]],
                        o_vmem.at[pl.ds(j * _W, _W)],
                        sem,
                    )
                    for j in range(_K)
                ]
                for c in copies:
                    c.wait()

            pl.run_scoped(scoped, pltpu.SemaphoreType.DMA)

        pltpu.emit_pipeline(
            body,
            grid=(N // (_K * _W),),
            in_specs=[pl.BlockSpec((_K, _W), index_map=lambda i: (i, 0))],
            out_specs=[pl.BlockSpec((_K * _W, D), index_map=lambda i: (i, 0))],
            core_axis_name=("core", "subcore"),
            dimension_semantics=(pltpu.PARALLEL,),
        )(i_hbm, o_hbm)

    out = gather_kernel(embeddings, idx)
    return out.reshape(B, H, D)
